# Initial kernel scaffold; baseline (speedup 1.0000x reference)
import jax
import jax.numpy as jnp
from jax.experimental import pallas as pl


def kernel(x, M1, M2):
    raise NotImplementedError("write your pallas kernel here")


# speedup vs baseline: 31.0626x; 31.0626x over previous
import jax
import jax.numpy as jnp
from jax.experimental import pallas as pl

NUM_NODE = 2048
RANK = 64
K_KEEP = int(0.01 * NUM_NODE * NUM_NODE)
NEG = -9000000000000000.0
_ROWS_PER_BLK = 128
_INF_BITS = 0x7F800000


def _gl_body(m1_ref, m2_ref, out_ref):
    adj = jnp.maximum(
        jax.lax.dot_general(
            m1_ref[...], m2_ref[...],
            dimension_numbers=(((1,), (1,)), ((), ())),
            preferred_element_type=jnp.float32,
        ),
        0.0,
    )

    def search(_, carry):
        lo, hi = carry
        mid = lo + (hi - lo) // 2
        bits = jax.lax.bitcast_convert_type(adj, jnp.int32)
        cnt = jnp.sum((bits > mid).astype(jnp.int32))
        big = cnt >= K_KEEP
        return jnp.where(big, mid, lo), jnp.where(big, hi, mid)

    _, kth_bits = jax.lax.fori_loop(
        0, 31, search, (jnp.int32(-1), jnp.int32(_INF_BITS))
    )

    for i in range(NUM_NODE // _ROWS_PER_BLK):
        blk = adj[i * _ROWS_PER_BLK:(i + 1) * _ROWS_PER_BLK, :]
        bbits = jax.lax.bitcast_convert_type(blk, jnp.int32)
        masked = jnp.where(bbits > kth_bits, blk, NEG)
        rowmax = jnp.max(masked, axis=1, keepdims=True)
        e = jnp.exp(masked - rowmax)
        p = e / jnp.sum(e, axis=1, keepdims=True)
        cols = jax.lax.broadcasted_iota(jnp.int32, (_ROWS_PER_BLK, NUM_NODE), 1)
        rows = jax.lax.broadcasted_iota(jnp.int32, (_ROWS_PER_BLK, NUM_NODE), 0)
        rows = rows + i * _ROWS_PER_BLK
        out_ref[i * _ROWS_PER_BLK:(i + 1) * _ROWS_PER_BLK, :] = jnp.where(
            rows == cols, 1.0, p
        )


def kernel(x, M1, M2):
    del x
    return pl.pallas_call(
        _gl_body,
        out_shape=jax.ShapeDtypeStruct((NUM_NODE, NUM_NODE), jnp.float32),
    )(M1, M2)
